# baseline (speedup 1.0000x reference)
import functools

import jax
import jax.numpy as jnp
from jax import lax
from jax.experimental import pallas as pl
from jax.experimental.pallas import tpu as pltpu
from jax.experimental.pallas import tpu_sc as plsc

VOCAB = 100000
D = 128
H1 = 256
H2 = 128
B = 4096
LQ = 20
LD = 200

NC = 2
NS = 16
NW = NC * NS
ROWS_PER_TILE = B // NW
DC = 100


def _scale_body(e_ref, w_ref, o_ref):
    o_ref[...] = e_ref[...] * w_ref[...]


def _scale_table(emb_table, w_table):
    blk = 20000
    grid = VOCAB // blk
    return pl.pallas_call(
        _scale_body,
        grid=(grid,),
        in_specs=[
            pl.BlockSpec((blk, D), lambda i: (i, 0)),
            pl.BlockSpec((blk, 1), lambda i: (i, 0)),
        ],
        out_specs=pl.BlockSpec((blk, D), lambda i: (i, 0)),
        out_shape=jax.ShapeDtypeStruct((VOCAB, D), jnp.float32),
    )(emb_table, w_table)


def _sc_pool_body(etab, doc3, q2, out_q, out_d,
                  idx_d, idx_q, d_rows0, q_rows0, d_rows1, q_rows1,
                  tq0, td0, tq1, td1, tq2, td2, tq3, td3,
                  sem0, sem1, sems0, sems1, sems2, sems3):
    wid = lax.axis_index("s") * NC + lax.axis_index("c")
    base = wid * ROWS_PER_TILE

    pltpu.sync_copy(doc3.at[pl.ds(base, ROWS_PER_TILE)], idx_d)
    pltpu.sync_copy(q2.at[pl.ds(base, ROWS_PER_TILE)], idx_q)

    gbufs = [(d_rows0, q_rows0, sem0), (d_rows1, q_rows1, sem1)]
    slots = [(tq0, td0, sems0), (tq1, td1, sems1),
             (tq2, td2, sems2), (tq3, td3, sems3)]

    def issue(b, d_rows, q_rows, sem):
        pltpu.async_copy(etab.at[idx_d.at[b, 0]], d_rows.at[pl.ds(0, DC)], sem)
        pltpu.async_copy(etab.at[idx_d.at[b, 1]], d_rows.at[pl.ds(DC, DC)], sem)
        pltpu.async_copy(etab.at[idx_q.at[b]], q_rows, sem)

    def wait(b, d_rows, q_rows, sem):
        pltpu.make_async_copy(etab.at[idx_d.at[b, 0]], d_rows.at[pl.ds(0, DC)], sem).wait()
        pltpu.make_async_copy(etab.at[idx_d.at[b, 1]], d_rows.at[pl.ds(DC, DC)], sem).wait()
        pltpu.make_async_copy(etab.at[idx_q.at[b]], q_rows, sem).wait()

    def wait_store(b, tq, td, sems):
        pltpu.make_async_copy(tq, out_q.at[pl.ds(base + b, 1)], sems).wait()
        pltpu.make_async_copy(td, out_d.at[pl.ds(base + b, 1)], sems).wait()

    def compute(b, d_rows, q_rows, tq, td, sems):
        U = 8
        def tok_body(i, accs):
            accs = list(accs)
            t0 = i * U
            for k in range(U):
                for c in range(8):
                    accs[c] = accs[c] + d_rows[t0 + k, pl.ds(c * 16, 16)]
            return tuple(accs)

        zero = jnp.zeros((16,), jnp.float32)
        accd = lax.fori_loop(0, LD // U, tok_body, (zero,) * 8)

        accq = [zero] * 8
        for t in range(LQ):
            for c in range(8):
                accq[c] = accq[c] + q_rows[t, pl.ds(c * 16, 16)]

        for c in range(8):
            td[0, pl.ds(c * 16, 16)] = accd[c]
            tq[0, pl.ds(c * 16, 16)] = accq[c]
        pltpu.async_copy(tq, out_q.at[pl.ds(base + b, 1)], sems)
        pltpu.async_copy(td, out_d.at[pl.ds(base + b, 1)], sems)

    issue(0, *gbufs[0])
    issue(1, *gbufs[1])

    def quad_body(g, carry):
        b0 = 4 * g
        for k in range(4):
            b = b0 + k
            d_rows, q_rows, gsem = gbufs[k % 2]
            tq, td, ssem = slots[k]
            wait(b, d_rows, q_rows, gsem)

            @pl.when(b >= 4)
            def _():
                wait_store(b - 4, tq, td, ssem)

            compute(b, d_rows, q_rows, tq, td, ssem)

            @pl.when(b + 2 < ROWS_PER_TILE)
            def _():
                issue(b + 2, d_rows, q_rows, gsem)
        return carry

    lax.fori_loop(0, ROWS_PER_TILE // 4, quad_body, 0)

    for k in range(4):
        tq, td, ssem = slots[k]
        wait_store(ROWS_PER_TILE - 4 + k, tq, td, ssem)


def _sc_pool(etab_scaled, doc3, q2):
    mesh = plsc.VectorSubcoreMesh(core_axis_name="c", subcore_axis_name="s")
    f = functools.partial(
        pl.kernel,
        mesh=mesh,
        out_type=[
            jax.ShapeDtypeStruct((B, D), jnp.float32),
            jax.ShapeDtypeStruct((B, D), jnp.float32),
        ],
        scratch_types=[
            pltpu.VMEM((ROWS_PER_TILE, 2, DC), jnp.int32),
            pltpu.VMEM((ROWS_PER_TILE, LQ), jnp.int32),
            pltpu.VMEM((LD, D), jnp.float32),
            pltpu.VMEM((LQ, D), jnp.float32),
            pltpu.VMEM((LD, D), jnp.float32),
            pltpu.VMEM((LQ, D), jnp.float32),
        ] + [pltpu.VMEM((1, D), jnp.float32)] * 8 + [
            pltpu.SemaphoreType.DMA,
        ] * 6,
    )(_sc_pool_body)
    return f(etab_scaled, doc3, q2)


def _mlp_body(pq_ref, pd_ref, lq_ref, ld_ref, w1_ref, b1_ref, w2_ref,
              b2_ref, w3_ref, b3_ref, o_ref):
    xq = pq_ref[...] / lq_ref[...]
    xd = pd_ref[...] / ld_ref[...]
    h1 = (jnp.dot(xq, w1_ref[0:D, :], preferred_element_type=jnp.float32)
          + jnp.dot(xd, w1_ref[D:2 * D, :], preferred_element_type=jnp.float32)
          + b1_ref[...])
    h1 = jnp.maximum(h1, 0.0)
    h2 = jnp.dot(h1, w2_ref[...], preferred_element_type=jnp.float32) + b2_ref[...]
    h2 = jnp.maximum(h2, 0.0)
    o_ref[...] = jnp.dot(h2, w3_ref[...], preferred_element_type=jnp.float32) + b3_ref[...]


def _mlp(pq, pd, lq, ld, W1, b1, W2, b2, W3, b3):
    return pl.pallas_call(
        _mlp_body,
        out_shape=jax.ShapeDtypeStruct((B, 1), jnp.float32),
    )(pq, pd, lq.reshape(B, 1), ld.reshape(B, 1), W1,
      b1.reshape(1, H1), W2, b2.reshape(1, H2), W3, b3.reshape(1, 1))


def kernel(q, doc, lengths_q, lengths_d, emb_table, w_table,
           W1, b1, W2, b2, W3, b3):
    q = q.astype(jnp.int32)
    doc = doc.astype(jnp.int32)
    etab_scaled = _scale_table(emb_table, w_table)
    doc3 = doc.reshape(B, 2, DC)
    pooled_q, pooled_d = _sc_pool(etab_scaled, doc3, q)
    out = _mlp(pooled_q, pooled_d, lengths_q, lengths_d,
               W1, b1, W2, b2, W3, b3)
    return jnp.squeeze(out, axis=1)


# speedup vs baseline: 1.0192x; 1.0192x over previous
import functools

import jax
import jax.numpy as jnp
from jax import lax
from jax.experimental import pallas as pl
from jax.experimental.pallas import tpu as pltpu
from jax.experimental.pallas import tpu_sc as plsc

VOCAB = 100000
D = 128
H1 = 256
H2 = 128
B = 4096
LQ = 20
LD = 200

NC = 2
NS = 16
NW = NC * NS
ROWS_PER_TILE = B // NW
DC = 100


def _scale_body(e_ref, w_ref, o_ref):
    o_ref[...] = e_ref[...] * w_ref[...]


def _scale_table(emb_table, w_table):
    blk = 10000
    grid = VOCAB // blk
    return pl.pallas_call(
        _scale_body,
        grid=(grid,),
        in_specs=[
            pl.BlockSpec((blk, D), lambda i: (i, 0)),
            pl.BlockSpec((blk, 1), lambda i: (i, 0)),
        ],
        out_specs=pl.BlockSpec((blk, D), lambda i: (i, 0)),
        out_shape=jax.ShapeDtypeStruct((VOCAB, D), jnp.float32),
    )(emb_table, w_table)


def _sc_pool_body(etab, doc3, q2, out_q, out_d,
                  idx_d, idx_q, d_rows0, q_rows0, d_rows1, q_rows1,
                  tq0, td0, tq1, td1, tq2, td2, tq3, td3,
                  sem0, sem1, sems0, sems1, sems2, sems3):
    wid = lax.axis_index("s") * NC + lax.axis_index("c")
    base = wid * ROWS_PER_TILE

    pltpu.sync_copy(doc3.at[pl.ds(base, ROWS_PER_TILE)], idx_d)
    pltpu.sync_copy(q2.at[pl.ds(base, ROWS_PER_TILE)], idx_q)

    gbufs = [(d_rows0, q_rows0, sem0), (d_rows1, q_rows1, sem1)]
    slots = [(tq0, td0, sems0), (tq1, td1, sems1),
             (tq2, td2, sems2), (tq3, td3, sems3)]

    def issue(b, d_rows, q_rows, sem):
        pltpu.async_copy(etab.at[idx_d.at[b, 0]], d_rows.at[pl.ds(0, DC)], sem)
        pltpu.async_copy(etab.at[idx_d.at[b, 1]], d_rows.at[pl.ds(DC, DC)], sem)
        pltpu.async_copy(etab.at[idx_q.at[b]], q_rows, sem)

    def wait0(b, d_rows, q_rows, sem):
        pltpu.make_async_copy(etab.at[idx_d.at[b, 0]], d_rows.at[pl.ds(0, DC)], sem).wait()

    def wait1(b, d_rows, q_rows, sem):
        pltpu.make_async_copy(etab.at[idx_d.at[b, 1]], d_rows.at[pl.ds(DC, DC)], sem).wait()
        pltpu.make_async_copy(etab.at[idx_q.at[b]], q_rows, sem).wait()

    def wait_store(b, tq, td, sems):
        pltpu.make_async_copy(tq, out_q.at[pl.ds(base + b, 1)], sems).wait()
        pltpu.make_async_copy(td, out_d.at[pl.ds(base + b, 1)], sems).wait()

    def accum_half(d_rows, base_t, accs_init):
        U = 10
        def tok_body(i, accs):
            accs = list(accs)
            t0 = base_t + i * U
            for k in range(U):
                for c in range(8):
                    accs[c] = accs[c] + d_rows[t0 + k, pl.ds(c * 16, 16)]
            return tuple(accs)
        return lax.fori_loop(0, DC // U, tok_body, accs_init)

    def compute_tail(b, d_rows, q_rows, tq, td, sems, accs_half):
        zero = jnp.zeros((16,), jnp.float32)
        accd = accum_half(d_rows, DC, accs_half)

        accq = [zero] * 8
        for t in range(LQ):
            for c in range(8):
                accq[c] = accq[c] + q_rows[t, pl.ds(c * 16, 16)]

        for c in range(8):
            td[0, pl.ds(c * 16, 16)] = accd[c]
            tq[0, pl.ds(c * 16, 16)] = accq[c]
        pltpu.async_copy(tq, out_q.at[pl.ds(base + b, 1)], sems)
        pltpu.async_copy(td, out_d.at[pl.ds(base + b, 1)], sems)

    issue(0, *gbufs[0])
    issue(1, *gbufs[1])

    def quad_body(g, carry):
        b0 = 4 * g
        zero = jnp.zeros((16,), jnp.float32)
        for k in range(4):
            b = b0 + k
            d_rows, q_rows, gsem = gbufs[k % 2]
            tq, td, ssem = slots[k]
            wait0(b, d_rows, q_rows, gsem)

            @pl.when(b >= 4)
            def _():
                wait_store(b - 4, tq, td, ssem)

            accs_half = accum_half(d_rows, 0, (zero,) * 8)
            wait1(b, d_rows, q_rows, gsem)
            compute_tail(b, d_rows, q_rows, tq, td, ssem, accs_half)

            @pl.when(b + 2 < ROWS_PER_TILE)
            def _():
                issue(b + 2, d_rows, q_rows, gsem)
        return carry

    lax.fori_loop(0, ROWS_PER_TILE // 4, quad_body, 0)

    for k in range(4):
        tq, td, ssem = slots[k]
        wait_store(ROWS_PER_TILE - 4 + k, tq, td, ssem)


def _sc_pool(etab_scaled, doc3, q2):
    mesh = plsc.VectorSubcoreMesh(core_axis_name="c", subcore_axis_name="s")
    f = functools.partial(
        pl.kernel,
        mesh=mesh,
        out_type=[
            jax.ShapeDtypeStruct((B, D), jnp.float32),
            jax.ShapeDtypeStruct((B, D), jnp.float32),
        ],
        scratch_types=[
            pltpu.VMEM((ROWS_PER_TILE, 2, DC), jnp.int32),
            pltpu.VMEM((ROWS_PER_TILE, LQ), jnp.int32),
            pltpu.VMEM((LD, D), jnp.float32),
            pltpu.VMEM((LQ, D), jnp.float32),
            pltpu.VMEM((LD, D), jnp.float32),
            pltpu.VMEM((LQ, D), jnp.float32),
        ] + [pltpu.VMEM((1, D), jnp.float32)] * 8 + [
            pltpu.SemaphoreType.DMA,
        ] * 6,
    )(_sc_pool_body)
    return f(etab_scaled, doc3, q2)


def _mlp_body(pq_ref, pd_ref, lq_ref, ld_ref, w1_ref, b1_ref, w2_ref,
              b2_ref, w3_ref, b3_ref, o_ref):
    xq = pq_ref[...] / lq_ref[...]
    xd = pd_ref[...] / ld_ref[...]
    h1 = (jnp.dot(xq, w1_ref[0:D, :], preferred_element_type=jnp.float32)
          + jnp.dot(xd, w1_ref[D:2 * D, :], preferred_element_type=jnp.float32)
          + b1_ref[...])
    h1 = jnp.maximum(h1, 0.0)
    h2 = jnp.dot(h1, w2_ref[...], preferred_element_type=jnp.float32) + b2_ref[...]
    h2 = jnp.maximum(h2, 0.0)
    o_ref[...] = jnp.dot(h2, w3_ref[...], preferred_element_type=jnp.float32) + b3_ref[...]


def _mlp(pq, pd, lq, ld, W1, b1, W2, b2, W3, b3):
    return pl.pallas_call(
        _mlp_body,
        out_shape=jax.ShapeDtypeStruct((B, 1), jnp.float32),
    )(pq, pd, lq.reshape(B, 1), ld.reshape(B, 1), W1,
      b1.reshape(1, H1), W2, b2.reshape(1, H2), W3, b3.reshape(1, 1))


def kernel(q, doc, lengths_q, lengths_d, emb_table, w_table,
           W1, b1, W2, b2, W3, b3):
    q = q.astype(jnp.int32)
    doc = doc.astype(jnp.int32)
    etab_scaled = _scale_table(emb_table, w_table)
    doc3 = doc.reshape(B, 2, DC)
    pooled_q, pooled_d = _sc_pool(etab_scaled, doc3, q)
    out = _mlp(pooled_q, pooled_d, lengths_q, lengths_d,
               W1, b1, W2, b2, W3, b3)
    return jnp.squeeze(out, axis=1)
